# baseline (speedup 1.0000x reference)
import functools

import jax
import jax.numpy as jnp
import numpy as np
from jax.experimental import pallas as pl
from jax.experimental.pallas import tpu as pltpu

B, C, SEQ, DM, K, E = 32, 64, 1024, 2048, 2, 4
PATCH = [96, 48, 24, 12]
NTOK = B * C
FREQ = SEQ // 2 + 1

TBLK = 512


def _cfg(p):
    s = p // 2
    pleft = p // 2
    pright = p - pleft
    win = SEQ // s + 1
    inner = DM // win
    return s, pleft, pright, win, inner


_n = np.arange(SEQ, dtype=np.int64)[:, None]
_k = np.arange(FREQ, dtype=np.int64)[None, :]
_ang = 2.0 * np.pi * ((_n * _k) % SEQ).astype(np.float64) / SEQ
_COS_NP = np.cos(_ang).astype(np.float32)
_SIN_NP = np.sin(_ang).astype(np.float32)

_HI = jax.lax.Precision.HIGHEST
_DEF = jax.lax.Precision.DEFAULT


def _dot(a, b, precision=_DEF):
    return jax.lax.dot_general(a, b, (((1,), (0,)), ((), ())),
                               precision=precision,
                               preferred_element_type=jnp.float32)


def _gelu(v):
    return 0.5 * v * (1.0 + jax.lax.erf(v * 0.7071067811865476))


def _gating_kernel(x_ref, cos_ref, sin_ref, wg1x_ref, wg1f_ref, bg1_ref,
                   wg2_ref, bg2_ref, gates_ref):
    x = x_ref[...]
    re = _dot(x, cos_ref[...], _HI)
    im = _dot(x, sin_ref[...], _HI)
    xf = jnp.sqrt(re * re + im * im)
    hg = _gelu(_dot(x, wg1x_ref[...]) + _dot(xf, wg1f_ref[...]) + bg1_ref[...])
    logits = _dot(hg, wg2_ref[...]) + bg2_ref[...]

    idx = jax.lax.broadcasted_iota(jnp.int32, (TBLK, E), 1)
    neg = jnp.float32(-1e30)
    m1 = jnp.max(logits, axis=1, keepdims=True)
    i1 = jnp.min(jnp.where(logits == m1, idx, E), axis=1, keepdims=True)
    l2 = jnp.where(idx == i1, neg, logits)
    m2 = jnp.max(l2, axis=1, keepdims=True)
    i2 = jnp.min(jnp.where(l2 == m2, idx, E), axis=1, keepdims=True)
    z = jnp.exp(m2 - m1)
    g1 = 1.0 / (1.0 + z)
    g2 = z / (1.0 + z)
    gates_ref[...] = g1 * (idx == i1) + g2 * (idx == i2)


WBS = [6, 11, 22, 43]


def _banded(W1, WB, s, p, inner):
    M = jnp.zeros(((WB + 1) * s, WB * inner), jnp.float32)
    for t in range(WB):
        M = M.at[t * s:t * s + p, t * inner:(t + 1) * inner].set(W1)
    return M


def _phase1(e, x, m_ref, b1t_ref, hf_ref):
    p = PATCH[e]
    s, pleft, pright, win, inner = _cfg(p)
    WB = WBS[e]
    xp = jnp.pad(x, ((0, 0), (pleft, pright)))
    g = 0
    while g * WB < win:
        w0 = g * WB
        nw = min(WB, win - w0)
        seg = xp[:, w0 * s:w0 * s + (nw + 1) * s]
        mm = m_ref[:(nw + 1) * s, :nw * inner]
        piece = _gelu(_dot(seg, mm) + b1t_ref[:, :nw * inner])
        hf_ref[:, w0 * inner:(w0 + nw) * inner] = piece.astype(hf_ref.dtype)
        g += 1


def _expert_pair_kernel(e0, e1, first, x_ref, gates_ref, *rest):
    if first:
        (m0, b1t0, w20, b20, m1, b1t1, w21, b21,
         out_ref, hf0, hf1) = rest
        prev = None
    else:
        (prev_ref, m0, b1t0, w20, b20, m1, b1t1, w21, b21,
         out_ref, hf0, hf1) = rest
        prev = prev_ref[...]
    x = x_ref[...]
    _phase1(e0, x, m0, b1t0, hf0)
    _phase1(e1, x, m1, b1t1, hf1)
    oa = _dot(hf0[...], w20[...]) + b20[...]
    ob = _dot(hf1[...], w21[...]) + b21[...]
    contrib = gates_ref[:, e0:e0 + 1] * oa + gates_ref[:, e1:e1 + 1] * ob
    out_ref[...] = contrib if prev is None else prev + contrib


def _full_spec(a):
    return pl.BlockSpec(a.shape, lambda i: tuple(0 for _ in a.shape))


@jax.jit
def kernel(x, W1_0, b1_0, W2_0, b2_0, W1_1, b1_1, W2_1, b2_1,
           W1_2, b1_2, W2_2, b2_2, W1_3, b1_3, W2_3, b2_3,
           Wg1, bg1, Wg2, bg2):
    x2d = x.reshape(NTOK, SEQ)
    wg1x = Wg1[:SEQ]
    wg1f = Wg1[SEQ:]
    r2 = lambda v: v.reshape(1, -1)

    grid = (NTOK // TBLK,)
    x_spec = pl.BlockSpec((TBLK, SEQ), lambda i: (i, 0))
    gates_spec = pl.BlockSpec((TBLK, E), lambda i: (i, 0))
    out_spec = pl.BlockSpec((TBLK, DM), lambda i: (i, 0))

    g_ops = [x2d, jnp.asarray(_COS_NP), jnp.asarray(_SIN_NP),
             wg1x, wg1f, r2(bg1), Wg2, r2(bg2)]
    gates = pl.pallas_call(
        _gating_kernel,
        grid=grid,
        in_specs=[x_spec] + [_full_spec(a) for a in g_ops[1:]],
        out_specs=gates_spec,
        out_shape=jax.ShapeDtypeStruct((NTOK, E), jnp.float32),
        compiler_params=pltpu.CompilerParams(
            vmem_limit_bytes=100 * 1024 * 1024),
    )(*g_ops)

    eweights = ((W1_0, b1_0, W2_0, b2_0), (W1_1, b1_1, W2_1, b2_1),
                (W1_2, b1_2, W2_2, b2_2), (W1_3, b1_3, W2_3, b2_3))
    out = None
    for (ea, eb) in ((0, 1), (2, 3)):
        first = out is None
        ops = [x2d, gates]
        specs = [x_spec, gates_spec]
        if not first:
            ops.append(out)
            specs.append(out_spec)
        scratch = []
        for e in (ea, eb):
            W1e, b1e, W2e, b2e = eweights[e]
            s, pleft, pright, win, inner = _cfg(PATCH[e])
            WB = WBS[e]
            Me = _banded(W1e, WB, s, PATCH[e], inner)
            b1t = jnp.tile(b1e, WB).reshape(1, -1)
            for a in (Me, b1t, W2e.astype(jnp.bfloat16), r2(b2e)):
                ops.append(a)
                specs.append(_full_spec(a))
            scratch.append(pltpu.VMEM((TBLK, win * inner), jnp.bfloat16))
        body = functools.partial(_expert_pair_kernel, ea, eb, first)
        out = pl.pallas_call(
            body,
            grid=grid,
            in_specs=specs,
            out_specs=out_spec,
            out_shape=jax.ShapeDtypeStruct((NTOK, DM), jnp.float32),
            scratch_shapes=scratch,
            input_output_aliases={} if first else {2: 0},
            compiler_params=pltpu.CompilerParams(
                vmem_limit_bytes=100 * 1024 * 1024),
        )(*ops)
    return (out.reshape(B, C, DM), jnp.float32(0.0))


# speedup vs baseline: 1.0115x; 1.0115x over previous
import functools

import jax
import jax.numpy as jnp
import numpy as np
from jax.experimental import pallas as pl
from jax.experimental.pallas import tpu as pltpu

B, C, SEQ, DM, K, E = 32, 64, 1024, 2048, 2, 4
PATCH = [96, 48, 24, 12]
NTOK = B * C
FREQ = SEQ // 2 + 1

TBLK = 256


def _cfg(p):
    s = p // 2
    pleft = p // 2
    pright = p - pleft
    win = SEQ // s + 1
    inner = DM // win
    return s, pleft, pright, win, inner


_n = np.arange(SEQ, dtype=np.int64)[:, None]
_k = np.arange(FREQ, dtype=np.int64)[None, :]
_ang = 2.0 * np.pi * ((_n * _k) % SEQ).astype(np.float64) / SEQ
_COS_NP = np.cos(_ang).astype(np.float32)
_SIN_NP = np.sin(_ang).astype(np.float32)

_HI = jax.lax.Precision.HIGHEST
_DEF = jax.lax.Precision.DEFAULT


def _dot(a, b, precision=_DEF):
    return jax.lax.dot_general(a, b, (((1,), (0,)), ((), ())),
                               precision=precision,
                               preferred_element_type=jnp.float32)


def _gelu(v):
    return 0.5 * v * (1.0 + jax.lax.erf(v * 0.7071067811865476))


def _gating_kernel(x_ref, cos_ref, sin_ref, wg1x_ref, wg1f_ref, bg1_ref,
                   wg2_ref, bg2_ref, gates_ref):
    x = x_ref[...]
    re = _dot(x, cos_ref[...], _HI)
    im = _dot(x, sin_ref[...], _HI)
    xf = jnp.sqrt(re * re + im * im)
    hg = _gelu(_dot(x, wg1x_ref[...]) + _dot(xf, wg1f_ref[...]) + bg1_ref[...])
    logits = _dot(hg, wg2_ref[...]) + bg2_ref[...]

    idx = jax.lax.broadcasted_iota(jnp.int32, (TBLK, E), 1)
    neg = jnp.float32(-1e30)
    m1 = jnp.max(logits, axis=1, keepdims=True)
    i1 = jnp.min(jnp.where(logits == m1, idx, E), axis=1, keepdims=True)
    l2 = jnp.where(idx == i1, neg, logits)
    m2 = jnp.max(l2, axis=1, keepdims=True)
    i2 = jnp.min(jnp.where(l2 == m2, idx, E), axis=1, keepdims=True)
    z = jnp.exp(m2 - m1)
    g1 = 1.0 / (1.0 + z)
    g2 = z / (1.0 + z)
    gates_ref[...] = g1 * (idx == i1) + g2 * (idx == i2)


WBS = [6, 11, 22, 43]


def _banded(W1, WB, s, p, inner):
    M = jnp.zeros(((WB + 1) * s, WB * inner), jnp.float32)
    for t in range(WB):
        M = M.at[t * s:t * s + p, t * inner:(t + 1) * inner].set(W1)
    return M


def _phase1(e, x, m_ref, b1t_ref, hf_ref):
    p = PATCH[e]
    s, pleft, pright, win, inner = _cfg(p)
    WB = WBS[e]
    xp = jnp.pad(x, ((0, 0), (pleft, pright)))
    g = 0
    while g * WB < win:
        w0 = g * WB
        nw = min(WB, win - w0)
        seg = xp[:, w0 * s:w0 * s + (nw + 1) * s]
        mm = m_ref[:(nw + 1) * s, :nw * inner]
        piece = _gelu(_dot(seg, mm) + b1t_ref[:, :nw * inner])
        hf_ref[:, w0 * inner:(w0 + nw) * inner] = piece
        g += 1


def _expert_pair_kernel(e0, e1, first, x_ref, gates_ref, *rest):
    if first:
        (m0, b1t0, w20, b20, m1, b1t1, w21, b21,
         out_ref, hf0, hf1) = rest
        prev = None
    else:
        (prev_ref, m0, b1t0, w20, b20, m1, b1t1, w21, b21,
         out_ref, hf0, hf1) = rest
        prev = prev_ref[...]
    x = x_ref[...]
    _phase1(e0, x, m0, b1t0, hf0)
    _phase1(e1, x, m1, b1t1, hf1)
    oa = _dot(hf0[...].astype(jnp.bfloat16),
              w20[...].astype(jnp.bfloat16)) + b20[...]
    ob = _dot(hf1[...].astype(jnp.bfloat16),
              w21[...].astype(jnp.bfloat16)) + b21[...]
    contrib = gates_ref[:, e0:e0 + 1] * oa + gates_ref[:, e1:e1 + 1] * ob
    out_ref[...] = contrib if prev is None else prev + contrib


def _full_spec(a):
    return pl.BlockSpec(a.shape, lambda i: tuple(0 for _ in a.shape))


@jax.jit
def kernel(x, W1_0, b1_0, W2_0, b2_0, W1_1, b1_1, W2_1, b2_1,
           W1_2, b1_2, W2_2, b2_2, W1_3, b1_3, W2_3, b2_3,
           Wg1, bg1, Wg2, bg2):
    x2d = x.reshape(NTOK, SEQ)
    wg1x = Wg1[:SEQ]
    wg1f = Wg1[SEQ:]
    r2 = lambda v: v.reshape(1, -1)

    grid = (NTOK // TBLK,)
    x_spec = pl.BlockSpec((TBLK, SEQ), lambda i: (i, 0))
    gates_spec = pl.BlockSpec((TBLK, E), lambda i: (i, 0))
    out_spec = pl.BlockSpec((TBLK, DM), lambda i: (i, 0))

    g_ops = [x2d, jnp.asarray(_COS_NP), jnp.asarray(_SIN_NP),
             wg1x, wg1f, r2(bg1), Wg2, r2(bg2)]
    gates = pl.pallas_call(
        _gating_kernel,
        grid=grid,
        in_specs=[x_spec] + [_full_spec(a) for a in g_ops[1:]],
        out_specs=gates_spec,
        out_shape=jax.ShapeDtypeStruct((NTOK, E), jnp.float32),
        compiler_params=pltpu.CompilerParams(
            vmem_limit_bytes=100 * 1024 * 1024),
    )(*g_ops)

    eweights = ((W1_0, b1_0, W2_0, b2_0), (W1_1, b1_1, W2_1, b2_1),
                (W1_2, b1_2, W2_2, b2_2), (W1_3, b1_3, W2_3, b2_3))
    out = None
    for (ea, eb) in ((0, 1), (2, 3)):
        first = out is None
        ops = [x2d, gates]
        specs = [x_spec, gates_spec]
        if not first:
            ops.append(out)
            specs.append(out_spec)
        scratch = []
        for e in (ea, eb):
            W1e, b1e, W2e, b2e = eweights[e]
            s, pleft, pright, win, inner = _cfg(PATCH[e])
            WB = WBS[e]
            Me = _banded(W1e, WB, s, PATCH[e], inner)
            b1t = jnp.tile(b1e, WB).reshape(1, -1)
            for a in (Me, b1t, W2e, r2(b2e)):
                ops.append(a)
                specs.append(_full_spec(a))
            scratch.append(pltpu.VMEM((TBLK, win * inner), jnp.float32))
        body = functools.partial(_expert_pair_kernel, ea, eb, first)
        out = pl.pallas_call(
            body,
            grid=grid,
            in_specs=specs,
            out_specs=out_spec,
            out_shape=jax.ShapeDtypeStruct((NTOK, DM), jnp.float32),
            scratch_shapes=scratch,
            input_output_aliases={} if first else {2: 0},
            compiler_params=pltpu.CompilerParams(
                vmem_limit_bytes=100 * 1024 * 1024),
        )(*ops)
    return (out.reshape(B, C, DM), jnp.float32(0.0))
